# baseline (speedup 1.0000x reference)
import functools

import jax
import jax.numpy as jnp
from jax import lax
from jax.experimental import pallas as pl
from jax.experimental.pallas import tpu as pltpu
from jax.experimental.pallas import tpu_sc as plsc

N = 10000
E = 320000
D_IN = 128
H = 256
D_OUT = 2 * H

NC = 2
NS = 16
NW = NC * NS
EPW = E // NW
K = 80
NCHUNK = EPW // K
G = 8
NGRP = 16
N_PAD = 10240
ROWS_PER_TILE = N_PAD // NS

NBUF = 4
SLAG = 3


def _sc_agg_body(src_hbm, dst_hbm, x_hbm, zeros_hbm, out_hbm,
                 agg, *rest):
    rows = rest[:NBUF]
    src_sl = rest[NBUF:NBUF + 2]
    dst_sl = rest[NBUF + 2:NBUF + 4]
    gsem = rest[NBUF + 4:NBUF + 4 + NBUF]
    ssem = rest[NBUF + 4 + NBUF:NBUF + 4 + 2 * NBUF]
    slabsem = rest[NBUF + 4 + 2 * NBUF:]

    c = lax.axis_index("c")
    s = lax.axis_index("s")
    wid = s * NC + c

    row0 = s * ROWS_PER_TILE

    def slab_load(g, p):
        off = pl.multiple_of(g * G, G)
        pltpu.async_copy(src_hbm.at[wid, pl.ds(off, G)], src_sl[p],
                         slabsem[p])
        pltpu.async_copy(dst_hbm.at[wid, pl.ds(off, G)], dst_sl[p],
                         slabsem[p])

    def slab_wait(p):
        pltpu.make_async_copy(src_hbm.at[wid, pl.ds(0, G)], src_sl[p],
                              slabsem[p]).wait()
        pltpu.make_async_copy(dst_hbm.at[wid, pl.ds(0, G)], dst_sl[p],
                              slabsem[p]).wait()

    def gather(u, p, r):
        pltpu.async_copy(x_hbm.at[src_sl[p].at[r]], rows[u], gsem[u])

    def gather_wait(u, p, r):
        pltpu.make_async_copy(x_hbm.at[src_sl[p].at[r]], rows[u],
                              gsem[u]).wait()

    def scat(u, p, r):
        pltpu.async_copy(rows[u], agg.at[dst_sl[p].at[r]], ssem[u],
                         add=True)

    def scat_wait(u, p, r):
        pltpu.make_async_copy(rows[u], agg.at[dst_sl[p].at[r]],
                              ssem[u]).wait()

    def upr(j):
        return j % NBUF, (j // G) % 2, j % G

    pltpu.async_copy(zeros_hbm.at[pl.ds(row0, ROWS_PER_TILE)],
                     agg.at[pl.ds(row0, ROWS_PER_TILE)], ssem[0])
    slab_load(0, 0)
    slab_load(1, 1)
    pltpu.make_async_copy(zeros_hbm.at[pl.ds(row0, ROWS_PER_TILE)],
                          agg.at[pl.ds(row0, ROWS_PER_TILE)], ssem[0]).wait()
    plsc.subcore_barrier()
    slab_wait(0)
    slab_wait(1)
    for j in range(SLAG):
        gather(*upr(j))
    gather_wait(*upr(0))
    scat(*upr(0))
    gather(*upr(SLAG))
    for js in range(1, G):
        gather_wait(*upr(js))
        scat(*upr(js))
        scat_wait(*upr(js - 1))
        gather(*upr(js + SLAG))

    def pair(h, carry):
        g1 = 2 * h + 1
        for off in range(16):
            if off == 1:
                slab_load(g1 + 1, 0)
            if off == 9:
                slab_load(g1 + 2, 1)
            if off == 5:
                slab_wait(0)
            if off == 13:
                slab_wait(1)
            u, pp, r = (8 + off) % NBUF, (1 + off // G) % 2, off % G
            us, ps, rs = ((8 + off - 1) % NBUF,
                          (1 + (off - 1) // G) % 2, (off - 1) % G)
            ug, pg, rg = ((8 + off + SLAG) % NBUF,
                          (1 + (off + SLAG) // G) % 2, (off + SLAG) % G)
            gather_wait(u, pp, r)
            scat(u, pp, r)
            scat_wait(us, ps, rs)
            gather(ug, pg, rg)
        return carry

    lax.fori_loop(0, 7, pair, 0)

    for js in range(120, NCHUNK - SLAG):
        gather_wait(*upr(js))
        scat(*upr(js))
        scat_wait(*upr(js - 1))
        gather(*upr(js + SLAG))
    for js in range(NCHUNK - SLAG, NCHUNK):
        gather_wait(*upr(js))
        scat(*upr(js))
        scat_wait(*upr(js - 1))
    scat_wait(*upr(NCHUNK - 1))

    plsc.subcore_barrier()

    pltpu.sync_copy(agg.at[pl.ds(row0, ROWS_PER_TILE)],
                    out_hbm.at[c, pl.ds(row0, ROWS_PER_TILE)])


_sc_agg = functools.partial(
    pl.kernel,
    out_type=jax.ShapeDtypeStruct((NC, N_PAD, D_IN), jnp.float32),
    mesh=plsc.VectorSubcoreMesh(core_axis_name="c", subcore_axis_name="s"),
    scratch_types=(
        [
            pltpu.VMEM_SHARED((N_PAD, D_IN), jnp.float32),
        ]
        + [pltpu.VMEM((K, D_IN), jnp.float32)] * NBUF
        + [pltpu.VMEM((G, K), jnp.int32)] * 2
        + [pltpu.VMEM((G, K), jnp.int32)] * 2
        + [pltpu.SemaphoreType.DMA] * (2 * NBUF)
        + [pltpu.SemaphoreType.DMA] * 2
    ),
)(_sc_agg_body)


def _tc_body(x_ref, aggs_ref, wr_ref, wn_ref, b_ref, r_ref, z_ref):
    a = aggs_ref[0] + aggs_ref[1]
    h = (
        jnp.dot(x_ref[...], wr_ref[...], preferred_element_type=jnp.float32)
        + jnp.dot(a, wn_ref[...], preferred_element_type=jnp.float32)
        + b_ref[...]
    )
    r_ref[...] = h[:, :H]
    z_ref[...] = h[:, H:]


def _tc_matmul(x, aggs, W_root, W_nbr, bvec):
    BLK = 10000
    return pl.pallas_call(
        _tc_body,
        grid=(N // BLK,),
        in_specs=[
            pl.BlockSpec((BLK, D_IN), lambda i: (i, 0)),
            pl.BlockSpec((NC, BLK, D_IN), lambda i: (0, i, 0)),
            pl.BlockSpec((D_IN, D_OUT), lambda i: (0, 0)),
            pl.BlockSpec((D_IN, D_OUT), lambda i: (0, 0)),
            pl.BlockSpec((1, D_OUT), lambda i: (0, 0)),
        ],
        out_specs=[
            pl.BlockSpec((BLK, H), lambda i: (i, 0)),
            pl.BlockSpec((BLK, H), lambda i: (i, 0)),
        ],
        out_shape=[
            jax.ShapeDtypeStruct((N, H), jnp.float32),
            jax.ShapeDtypeStruct((N, H), jnp.float32),
        ],
    )(x, aggs, W_root, W_nbr, bvec)


@jax.jit
def kernel(x, edge_index, W_root, W_nbr, b_gnn, bias):
    src = edge_index[0].reshape(NW, NCHUNK, K)
    dst = edge_index[1].reshape(NW, NCHUNK, K)
    pad = ((0, 0), (0, NGRP * G - NCHUNK), (0, 0))
    src = jnp.pad(src, pad)
    dst = jnp.pad(dst, pad)
    zeros = jnp.zeros((N_PAD, D_IN), jnp.float32)
    aggs = _sc_agg(src, dst, x, zeros)
    bvec = (b_gnn + bias).reshape(1, D_OUT)
    r, z = _tc_matmul(x, aggs, W_root, W_nbr, bvec)
    return (r, z)


# speedup vs baseline: 1.1015x; 1.1015x over previous
import functools

import jax
import jax.numpy as jnp
from jax import lax
from jax.experimental import pallas as pl
from jax.experimental.pallas import tpu as pltpu
from jax.experimental.pallas import tpu_sc as plsc

N = 10000
E = 320000
D_IN = 128
H = 256
D_OUT = 2 * H

NC = 2
NS = 16
NW = NC * NS
EPW = E // NW
K = 80
NCHUNK = EPW // K
G = 8
NGRP = 16
N_PAD = 10240
ROWS_PER_TILE = N_PAD // NS

NBUF = 4
SLAG = 3


def _sc_agg_body(src_hbm, dst_hbm, x_hbm, zeros_hbm, out_hbm,
                 agg, *rest):
    rows = rest[:NBUF]
    src_sl = rest[NBUF:NBUF + 2]
    dst_sl = rest[NBUF + 2:NBUF + 4]
    gsem = rest[NBUF + 4:NBUF + 4 + NBUF]
    ssem = rest[NBUF + 4 + NBUF:NBUF + 4 + 2 * NBUF]
    slabsem = rest[NBUF + 4 + 2 * NBUF:]

    c = lax.axis_index("c")
    s = lax.axis_index("s")
    wid = s * NC + c

    row0 = s * ROWS_PER_TILE

    def slab_load(g, p):
        off = pl.multiple_of(g * G, G)
        pltpu.async_copy(src_hbm.at[wid, pl.ds(off, G)], src_sl[p],
                         slabsem[p])
        pltpu.async_copy(dst_hbm.at[wid, pl.ds(off, G)], dst_sl[p],
                         slabsem[p])

    def slab_wait(p):
        pltpu.make_async_copy(src_hbm.at[wid, pl.ds(0, G)], src_sl[p],
                              slabsem[p]).wait()
        pltpu.make_async_copy(dst_hbm.at[wid, pl.ds(0, G)], dst_sl[p],
                              slabsem[p]).wait()

    def gather(u, p, r):
        pltpu.async_copy(x_hbm.at[src_sl[p].at[r]], rows[u], gsem[u])

    def gather_wait(u, p, r):
        pltpu.make_async_copy(x_hbm.at[src_sl[p].at[r]], rows[u],
                              gsem[u]).wait()

    def scat(u, p, r):
        pltpu.async_copy(rows[u], agg.at[dst_sl[p].at[r]], ssem[u],
                         add=True)

    def scat_wait(u, p, r):
        pltpu.make_async_copy(rows[u], agg.at[dst_sl[p].at[r]],
                              ssem[u]).wait()

    def upr(j):
        return j % NBUF, (j // G) % 2, j % G

    pltpu.async_copy(zeros_hbm.at[pl.ds(row0, ROWS_PER_TILE)],
                     agg.at[pl.ds(row0, ROWS_PER_TILE)], ssem[0])
    slab_load(0, 0)
    slab_load(1, 1)
    pltpu.make_async_copy(zeros_hbm.at[pl.ds(row0, ROWS_PER_TILE)],
                          agg.at[pl.ds(row0, ROWS_PER_TILE)], ssem[0]).wait()
    plsc.subcore_barrier()
    slab_wait(0)
    slab_wait(1)
    for j in range(SLAG):
        gather(*upr(j))
    gather_wait(*upr(0))
    scat(*upr(0))
    gather(*upr(SLAG))
    for js in range(1, G):
        scat_wait(*upr(js - 1))
        gather(*upr(js + SLAG))
        gather_wait(*upr(js))
        scat(*upr(js))

    def pair(h, carry):
        g1 = 2 * h + 1
        for off in range(16):
            if off == 1:
                slab_load(g1 + 1, 0)
            if off == 9:
                slab_load(g1 + 2, 1)
            if off == 5:
                slab_wait(0)
            if off == 13:
                slab_wait(1)
            u, pp, r = (8 + off) % NBUF, (1 + off // G) % 2, off % G
            us, ps, rs = ((8 + off - 1) % NBUF,
                          (1 + (off - 1) // G) % 2, (off - 1) % G)
            ug, pg, rg = ((8 + off + SLAG) % NBUF,
                          (1 + (off + SLAG) // G) % 2, (off + SLAG) % G)
            scat_wait(us, ps, rs)
            gather(ug, pg, rg)
            gather_wait(u, pp, r)
            scat(u, pp, r)
        return carry

    lax.fori_loop(0, 7, pair, 0)

    for js in range(120, NCHUNK - SLAG):
        scat_wait(*upr(js - 1))
        gather(*upr(js + SLAG))
        gather_wait(*upr(js))
        scat(*upr(js))
    for js in range(NCHUNK - SLAG, NCHUNK):
        gather_wait(*upr(js))
        scat(*upr(js))
        scat_wait(*upr(js - 1))
    scat_wait(*upr(NCHUNK - 1))

    plsc.subcore_barrier()

    pltpu.sync_copy(agg.at[pl.ds(row0, ROWS_PER_TILE)],
                    out_hbm.at[c, pl.ds(row0, ROWS_PER_TILE)])


_sc_agg = functools.partial(
    pl.kernel,
    out_type=jax.ShapeDtypeStruct((NC, N_PAD, D_IN), jnp.float32),
    mesh=plsc.VectorSubcoreMesh(core_axis_name="c", subcore_axis_name="s"),
    scratch_types=(
        [
            pltpu.VMEM_SHARED((N_PAD, D_IN), jnp.float32),
        ]
        + [pltpu.VMEM((K, D_IN), jnp.float32)] * NBUF
        + [pltpu.VMEM((G, K), jnp.int32)] * 2
        + [pltpu.VMEM((G, K), jnp.int32)] * 2
        + [pltpu.SemaphoreType.DMA] * (2 * NBUF)
        + [pltpu.SemaphoreType.DMA] * 2
    ),
)(_sc_agg_body)


def _tc_body(x_ref, aggs_ref, wr_ref, wn_ref, b_ref, r_ref, z_ref):
    a = aggs_ref[0] + aggs_ref[1]
    h = (
        jnp.dot(x_ref[...], wr_ref[...], preferred_element_type=jnp.float32)
        + jnp.dot(a, wn_ref[...], preferred_element_type=jnp.float32)
        + b_ref[...]
    )
    r_ref[...] = h[:, :H]
    z_ref[...] = h[:, H:]


def _tc_matmul(x, aggs, W_root, W_nbr, bvec):
    BLK = 5000
    return pl.pallas_call(
        _tc_body,
        grid=(N // BLK,),
        in_specs=[
            pl.BlockSpec((BLK, D_IN), lambda i: (i, 0)),
            pl.BlockSpec((NC, BLK, D_IN), lambda i: (0, i, 0)),
            pl.BlockSpec((D_IN, D_OUT), lambda i: (0, 0)),
            pl.BlockSpec((D_IN, D_OUT), lambda i: (0, 0)),
            pl.BlockSpec((1, D_OUT), lambda i: (0, 0)),
        ],
        out_specs=[
            pl.BlockSpec((BLK, H), lambda i: (i, 0)),
            pl.BlockSpec((BLK, H), lambda i: (i, 0)),
        ],
        out_shape=[
            jax.ShapeDtypeStruct((N, H), jnp.float32),
            jax.ShapeDtypeStruct((N, H), jnp.float32),
        ],
    )(x, aggs, W_root, W_nbr, bvec)


@jax.jit
def kernel(x, edge_index, W_root, W_nbr, b_gnn, bias):
    src = edge_index[0].reshape(NW, NCHUNK, K)
    dst = edge_index[1].reshape(NW, NCHUNK, K)
    pad = ((0, 0), (0, NGRP * G - NCHUNK), (0, 0))
    src = jnp.pad(src, pad)
    dst = jnp.pad(dst, pad)
    zeros = jnp.zeros((N_PAD, D_IN), jnp.float32)
    aggs = _sc_agg(src, dst, x, zeros)
    bvec = (b_gnn + bias).reshape(1, D_OUT)
    r, z = _tc_matmul(x, aggs, W_root, W_nbr, bvec)
    return (r, z)


# speedup vs baseline: 1.1497x; 1.0437x over previous
import functools

import jax
import jax.numpy as jnp
from jax import lax
from jax.experimental import pallas as pl
from jax.experimental.pallas import tpu as pltpu
from jax.experimental.pallas import tpu_sc as plsc

N = 10000
E = 320000
D_IN = 128
H = 256
D_OUT = 2 * H

NC = 2
NS = 16
NW = NC * NS
EPW = E // NW
K = 80
NCHUNK = EPW // K
G = 8
NGRP = 16
N_PAD = 10240
ROWS_PER_TILE = N_PAD // NS

NBUF = 4
SLAG = 3


def _sc_agg_body(src_hbm, dst_hbm, x_hbm, out_hbm,
                 agg, *rest):
    rows = rest[:NBUF]
    src_sl = rest[NBUF:NBUF + 2]
    dst_sl = rest[NBUF + 2:NBUF + 4]
    gsem = rest[NBUF + 4:NBUF + 4 + NBUF]
    ssem = rest[NBUF + 4 + NBUF:NBUF + 4 + 2 * NBUF]
    slabsem = rest[NBUF + 4 + 2 * NBUF:]

    c = lax.axis_index("c")
    s = lax.axis_index("s")
    wid = s * NC + c

    row0 = s * ROWS_PER_TILE

    def slab_load(g, p):
        off = pl.multiple_of(g * G, G)
        pltpu.async_copy(src_hbm.at[wid, pl.ds(off, G)], src_sl[p],
                         slabsem[p])
        pltpu.async_copy(dst_hbm.at[wid, pl.ds(off, G)], dst_sl[p],
                         slabsem[p])

    def slab_wait(p):
        pltpu.make_async_copy(src_hbm.at[wid, pl.ds(0, G)], src_sl[p],
                              slabsem[p]).wait()
        pltpu.make_async_copy(dst_hbm.at[wid, pl.ds(0, G)], dst_sl[p],
                              slabsem[p]).wait()

    def gather(u, p, r):
        pltpu.async_copy(x_hbm.at[src_sl[p].at[r]], rows[u], gsem[u])

    def gather_wait(u, p, r):
        pltpu.make_async_copy(x_hbm.at[src_sl[p].at[r]], rows[u],
                              gsem[u]).wait()

    def scat(u, p, r):
        pltpu.async_copy(rows[u], agg.at[dst_sl[p].at[r]], ssem[u],
                         add=True)

    def scat_wait(u, p, r):
        pltpu.make_async_copy(rows[u], agg.at[dst_sl[p].at[r]],
                              ssem[u]).wait()

    def upr(j):
        return j % NBUF, (j // G) % 2, j % G

    def zbody(i, carry):
        for cc in range(D_IN // 16):
            rows[0][i, pl.ds(cc * 16, 16)] = jnp.zeros((16,), jnp.float32)
        return carry

    lax.fori_loop(0, K, zbody, 0)
    zsems = list(gsem) + list(ssem)
    for t in range(ROWS_PER_TILE // K):
        pltpu.async_copy(rows[0], agg.at[pl.ds(row0 + K * t, K)],
                         zsems[t])
    slab_load(0, 0)
    slab_load(1, 1)
    for t in range(ROWS_PER_TILE // K):
        pltpu.make_async_copy(rows[0], agg.at[pl.ds(row0 + K * t, K)],
                              zsems[t]).wait()
    plsc.subcore_barrier()
    slab_wait(0)
    slab_wait(1)
    for j in range(SLAG):
        gather(*upr(j))
    gather_wait(*upr(0))
    scat(*upr(0))
    gather(*upr(SLAG))
    for js in range(1, G):
        scat_wait(*upr(js - 1))
        gather(*upr(js + SLAG))
        gather_wait(*upr(js))
        scat(*upr(js))

    def pair(h, carry):
        g1 = 2 * h + 1
        for off in range(16):
            if off == 1:
                slab_load(g1 + 1, 0)
            if off == 9:
                slab_load(g1 + 2, 1)
            if off == 5:
                slab_wait(0)
            if off == 13:
                slab_wait(1)
            u, pp, r = (8 + off) % NBUF, (1 + off // G) % 2, off % G
            us, ps, rs = ((8 + off - 1) % NBUF,
                          (1 + (off - 1) // G) % 2, (off - 1) % G)
            ug, pg, rg = ((8 + off + SLAG) % NBUF,
                          (1 + (off + SLAG) // G) % 2, (off + SLAG) % G)
            scat_wait(us, ps, rs)
            gather(ug, pg, rg)
            gather_wait(u, pp, r)
            scat(u, pp, r)
        return carry

    lax.fori_loop(0, 7, pair, 0)

    for js in range(120, NCHUNK - SLAG):
        scat_wait(*upr(js - 1))
        gather(*upr(js + SLAG))
        gather_wait(*upr(js))
        scat(*upr(js))
    for js in range(NCHUNK - SLAG, NCHUNK):
        gather_wait(*upr(js))
        scat(*upr(js))
        scat_wait(*upr(js - 1))
    scat_wait(*upr(NCHUNK - 1))

    plsc.subcore_barrier()

    pltpu.sync_copy(agg.at[pl.ds(row0, ROWS_PER_TILE)],
                    out_hbm.at[c, pl.ds(row0, ROWS_PER_TILE)])


_sc_agg = functools.partial(
    pl.kernel,
    out_type=jax.ShapeDtypeStruct((NC, N_PAD, D_IN), jnp.float32),
    mesh=plsc.VectorSubcoreMesh(core_axis_name="c", subcore_axis_name="s"),
    scratch_types=(
        [
            pltpu.VMEM_SHARED((N_PAD, D_IN), jnp.float32),
        ]
        + [pltpu.VMEM((K, D_IN), jnp.float32)] * NBUF
        + [pltpu.VMEM((G, K), jnp.int32)] * 2
        + [pltpu.VMEM((G, K), jnp.int32)] * 2
        + [pltpu.SemaphoreType.DMA] * (2 * NBUF)
        + [pltpu.SemaphoreType.DMA] * 2
    ),
)(_sc_agg_body)


def _tc_body(x_ref, aggs_ref, wr_ref, wn_ref, b_ref, r_ref, z_ref):
    a = aggs_ref[0] + aggs_ref[1]
    h = (
        jnp.dot(x_ref[...], wr_ref[...], preferred_element_type=jnp.float32)
        + jnp.dot(a, wn_ref[...], preferred_element_type=jnp.float32)
        + b_ref[...]
    )
    r_ref[...] = h[:, :H]
    z_ref[...] = h[:, H:]


def _tc_matmul(x, aggs, W_root, W_nbr, bvec):
    BLK = 5000
    return pl.pallas_call(
        _tc_body,
        grid=(N // BLK,),
        in_specs=[
            pl.BlockSpec((BLK, D_IN), lambda i: (i, 0)),
            pl.BlockSpec((NC, BLK, D_IN), lambda i: (0, i, 0)),
            pl.BlockSpec((D_IN, D_OUT), lambda i: (0, 0)),
            pl.BlockSpec((D_IN, D_OUT), lambda i: (0, 0)),
            pl.BlockSpec((1, D_OUT), lambda i: (0, 0)),
        ],
        out_specs=[
            pl.BlockSpec((BLK, H), lambda i: (i, 0)),
            pl.BlockSpec((BLK, H), lambda i: (i, 0)),
        ],
        out_shape=[
            jax.ShapeDtypeStruct((N, H), jnp.float32),
            jax.ShapeDtypeStruct((N, H), jnp.float32),
        ],
    )(x, aggs, W_root, W_nbr, bvec)


@jax.jit
def kernel(x, edge_index, W_root, W_nbr, b_gnn, bias):
    src = edge_index[0].reshape(NW, NCHUNK, K)
    dst = edge_index[1].reshape(NW, NCHUNK, K)
    pad = ((0, 0), (0, NGRP * G - NCHUNK), (0, 0))
    src = jnp.pad(src, pad)
    dst = jnp.pad(dst, pad)
    aggs = _sc_agg(src, dst, x)
    bvec = (b_gnn + bias).reshape(1, D_OUT)
    r, z = _tc_matmul(x, aggs, W_root, W_nbr, bvec)
    return (r, z)
